# baseline (speedup 1.0000x reference)
import functools

import jax
import jax.numpy as jnp
from jax import lax
from jax.experimental import pallas as pl
from jax.experimental.pallas import tpu as pltpu
from jax.experimental.pallas import tpu_sc as plsc

_BATCH = 16384
_HIST = 50
_EMBED = 32

_NC = 2
_NS = 16
_NW = _NC * _NS
_ROWS_PER_W = _BATCH // _NW
_PADW = 128
_HB = _ROWS_PER_W // 2
_NBUF = 3


def _build_gather():
    mesh = plsc.VectorSubcoreMesh(core_axis_name="c", subcore_axis_name="s")

    @functools.partial(
        pl.kernel,
        out_type=jax.ShapeDtypeStruct((_BATCH, _HIST, _EMBED), jnp.float32),
        mesh=mesh,
        scratch_types=[
            pltpu.VMEM((_HIST, _ROWS_PER_W), jnp.int32),
            pltpu.VMEM((_NBUF, _HB, _PADW), jnp.float32),
            pltpu.SemaphoreType.DMA,
            pltpu.SemaphoreType.DMA,
        ],
        compiler_params=pltpu.CompilerParams(use_tc_tiling_on_sc=False),
    )
    def gather_kernel(xt_hbm, table_hbm, out_hbm, idx_v, rows_v, gsem, ssem):
        wid = lax.axis_index("s") * _NC + lax.axis_index("c")
        base = wid * _ROWS_PER_W

        pltpu.sync_copy(xt_hbm.at[:, pl.ds(base, _ROWS_PER_W)], idx_v)

        _NCH = _HIST * 2

        def start_gather(c):
            h, half = c // 2, c % 2
            return pltpu.async_copy(
                table_hbm.at[idx_v.at[h, pl.ds(half * _HB, _HB)]],
                rows_v.at[c % _NBUF], gsem)

        def start_store(c):
            h, half = c // 2, c % 2
            return pltpu.async_copy(
                rows_v.at[c % _NBUF, :, pl.ds(0, _EMBED)],
                out_hbm.at[pl.ds(base + half * _HB, _HB), h], ssem)

        gathers = [start_gather(b) for b in range(_NBUF)]
        stores = []
        for c in range(_NCH):
            gathers[c].wait()
            stores.append(start_store(c))
            nxt = c + _NBUF
            if nxt < _NCH:
                stores[c].wait()
                gathers.append(start_gather(nxt))
        for c in range(_NCH - _NBUF, _NCH):
            stores[c].wait()

    return gather_kernel


_GATHER = _build_gather()


@jax.jit
def kernel(x, table):
    tp = jnp.pad(table, ((0, 0), (0, _PADW - _EMBED)))
    return _GATHER(x.T.astype(jnp.int32), tp)


# speedup vs baseline: 1.0952x; 1.0952x over previous
import functools

import jax
import jax.numpy as jnp
from jax import lax
from jax.experimental import pallas as pl
from jax.experimental.pallas import tpu as pltpu
from jax.experimental.pallas import tpu_sc as plsc

_BATCH = 16384
_HIST = 50
_EMBED = 32

_NC = 2
_NS = 16
_NW = _NC * _NS
_ROWS_PER_W = _BATCH // _NW
_NBUF = 4


def _build_gather():
    mesh = plsc.VectorSubcoreMesh(core_axis_name="c", subcore_axis_name="s")

    @functools.partial(
        pl.kernel,
        out_type=jax.ShapeDtypeStruct((_BATCH, _HIST, _EMBED), jnp.float32),
        mesh=mesh,
        scratch_types=[
            pltpu.VMEM((_HIST, _ROWS_PER_W), jnp.int32),
            pltpu.VMEM((_NBUF, _ROWS_PER_W, _EMBED), jnp.float32),
            pltpu.SemaphoreType.DMA,
            pltpu.SemaphoreType.DMA,
        ],
        compiler_params=pltpu.CompilerParams(use_tc_tiling_on_sc=False),
    )
    def gather_kernel(xt_hbm, table_hbm, out_hbm, idx_v, rows_v, gsem, ssem):
        wid = lax.axis_index("s") * _NC + lax.axis_index("c")
        base = wid * _ROWS_PER_W

        pltpu.sync_copy(xt_hbm.at[:, pl.ds(base, _ROWS_PER_W)], idx_v)

        def start_gather(h):
            return pltpu.async_copy(
                table_hbm.at[idx_v.at[h]], rows_v.at[h % _NBUF], gsem)

        def start_store(h):
            return pltpu.async_copy(
                rows_v.at[h % _NBUF],
                out_hbm.at[pl.ds(base, _ROWS_PER_W), h], ssem)

        gathers = [start_gather(b) for b in range(_NBUF)]
        stores = []
        for h in range(_HIST):
            gathers[h].wait()
            stores.append(start_store(h))
            nxt = h + _NBUF
            if nxt < _HIST:
                stores[h].wait()
                gathers.append(start_gather(nxt))
        for h in range(_HIST - _NBUF, _HIST):
            stores[h].wait()

    return gather_kernel


_GATHER = _build_gather()


@jax.jit
def kernel(x, table):
    return _GATHER(x.T.astype(jnp.int32), table)
